# baseline (speedup 1.0000x reference)
import functools

import jax
import jax.numpy as jnp
from jax import lax
from jax.experimental import pallas as pl
from jax.experimental.pallas import tpu as pltpu
from jax.experimental.pallas import tpu_sc as plsc

B = 128
N = 100000
W = 4096
NB = (N + W - 1) // W


def _gather_v(flat_x, flat_idx):
    mesh = plsc.VectorSubcoreMesh(core_axis_name="c", subcore_axis_name="s")

    @functools.partial(
        pl.kernel,
        mesh=mesh,
        out_type=jax.ShapeDtypeStruct((B,), jnp.float32),
        scratch_types=[
            pltpu.VMEM((B,), jnp.int32),
            pltpu.VMEM((B,), jnp.float32),
            pltpu.SemaphoreType.DMA,
        ],
    )
    def gather_kernel(x_hbm, idx_hbm, v_hbm, idx_v, vals_v, sem):
        cid = lax.axis_index("c")
        sid = lax.axis_index("s")

        @pl.when(jnp.logical_and(cid == 0, sid == 0))
        def _():
            pltpu.sync_copy(idx_hbm, idx_v)
            pltpu.async_copy(x_hbm.at[idx_v], vals_v, sem).wait()
            pltpu.sync_copy(vals_v, v_hbm)

    return gather_kernel(flat_x, flat_idx)


def _count_kernel(x_ref, v_ref, t_ref, out1_ref, out5_ref, acc_ref):
    j = pl.program_id(0)

    @pl.when(j == 0)
    def _():
        acc_ref[...] = jnp.zeros_like(acc_ref)

    x = x_ref[...]
    v = v_ref[...]
    tl = t_ref[...] - j * W
    li = lax.broadcasted_iota(jnp.int32, (B, W), 1)
    eq_before = (x == v) & (li < tl)
    gt = x > v

    @pl.when(j < NB - 1)
    def _():
        hit = gt | eq_before
        acc_ref[...] += jnp.sum(hit.astype(jnp.int32), axis=1, keepdims=True)

    @pl.when(j == NB - 1)
    def _():
        hit = (gt & (li < (N - j * W))) | eq_before
        rank = acc_ref[...] + jnp.sum(hit.astype(jnp.int32), axis=1, keepdims=True)
        out1_ref[0, 0] = jnp.sum((rank < 1).astype(jnp.float32)) * (100.0 / B)
        out5_ref[0, 0] = jnp.sum((rank < 5).astype(jnp.float32)) * (100.0 / B)


def _count_ranks(x, v2, t2):
    return pl.pallas_call(
        _count_kernel,
        grid=(NB,),
        in_specs=[
            pl.BlockSpec((B, W), lambda j: (0, j)),
            pl.BlockSpec((B, 1), lambda j: (0, 0)),
            pl.BlockSpec((B, 1), lambda j: (0, 0)),
        ],
        out_specs=[
            pl.BlockSpec(memory_space=pltpu.SMEM),
            pl.BlockSpec(memory_space=pltpu.SMEM),
        ],
        out_shape=[
            jax.ShapeDtypeStruct((1, 1), jnp.float32),
            jax.ShapeDtypeStruct((1, 1), jnp.float32),
        ],
        scratch_shapes=[pltpu.VMEM((B, 1), jnp.int32)],
        compiler_params=pltpu.CompilerParams(
            dimension_semantics=("arbitrary",)),
    )(x, v2, t2)


def kernel(output, target):
    t32 = target.astype(jnp.int32)
    flat_idx = jnp.arange(B, dtype=jnp.int32) * N + t32
    v = _gather_v(output.reshape(-1), flat_idx)
    r1, r5 = _count_ranks(output, v.reshape(B, 1), t32.reshape(B, 1))
    return (r1.reshape(1), r5.reshape(1))


# speedup vs baseline: 2.0262x; 2.0262x over previous
import functools

import jax
import jax.numpy as jnp
from jax import lax
from jax.experimental import pallas as pl
from jax.experimental.pallas import tpu as pltpu
from jax.experimental.pallas import tpu_sc as plsc

B = 128
N = 100000
W = 4096
NB = (N + W - 1) // W


def _gather_v(flat_x, flat_idx):
    mesh = plsc.VectorSubcoreMesh(core_axis_name="c", subcore_axis_name="s")

    @functools.partial(
        pl.kernel,
        mesh=mesh,
        out_type=jax.ShapeDtypeStruct((B,), jnp.float32),
        scratch_types=[
            pltpu.VMEM((B,), jnp.int32),
            pltpu.VMEM((B,), jnp.float32),
            pltpu.SemaphoreType.DMA,
        ],
    )
    def gather_kernel(x_hbm, idx_hbm, v_hbm, idx_v, vals_v, sem):
        cid = lax.axis_index("c")
        sid = lax.axis_index("s")

        @pl.when(jnp.logical_and(cid == 0, sid == 0))
        def _():
            pltpu.sync_copy(idx_hbm, idx_v)
            pltpu.async_copy(x_hbm.at[idx_v], vals_v, sem).wait()
            pltpu.sync_copy(vals_v, v_hbm)

    return gather_kernel(flat_x, flat_idx)


def _count_kernel(x_ref, v_ref, t_ref, out1_ref, out5_ref, acc_ref):
    j = pl.program_id(0)

    @pl.when(j == 0)
    def _():
        acc_ref[...] = jnp.zeros_like(acc_ref)

    x = x_ref[...]
    v = v_ref[...]
    tl = t_ref[...] - j * W
    li = lax.broadcasted_iota(jnp.int32, (B, W), 1)
    eq_before = (x == v) & (li < tl)
    gt = x > v

    @pl.when(j < NB - 1)
    def _():
        hit = gt | eq_before
        acc_ref[...] += jnp.sum(hit.astype(jnp.int32), axis=1, keepdims=True)

    @pl.when(j == NB - 1)
    def _():
        hit = (gt & (li < (N - j * W))) | eq_before
        rank = acc_ref[...] + jnp.sum(hit.astype(jnp.int32), axis=1, keepdims=True)
        out1_ref[0, 0] = jnp.sum((rank < 1).astype(jnp.float32)) * (100.0 / B)
        out5_ref[0, 0] = jnp.sum((rank < 5).astype(jnp.float32)) * (100.0 / B)


def _count_ranks(x, v2, t2):
    return pl.pallas_call(
        _count_kernel,
        grid=(NB,),
        in_specs=[
            pl.BlockSpec((B, W), lambda j: (0, j)),
            pl.BlockSpec((B, 1), lambda j: (0, 0)),
            pl.BlockSpec((B, 1), lambda j: (0, 0)),
        ],
        out_specs=[
            pl.BlockSpec(memory_space=pltpu.SMEM),
            pl.BlockSpec(memory_space=pltpu.SMEM),
        ],
        out_shape=[
            jax.ShapeDtypeStruct((1, 1), jnp.float32),
            jax.ShapeDtypeStruct((1, 1), jnp.float32),
        ],
        scratch_shapes=[pltpu.VMEM((B, 1), jnp.int32)],
        compiler_params=pltpu.CompilerParams(
            dimension_semantics=("arbitrary",)),
    )(x, v2, t2)


def kernel(output, target):
    t32 = target.astype(jnp.int32)
    v = jnp.take_along_axis(output, t32[:, None], axis=1)
    r1, r5 = _count_ranks(output, v, t32.reshape(B, 1))
    return (r1.reshape(1), r5.reshape(1))
